# baseline (speedup 1.0000x reference)
import jax
import jax.numpy as jnp
from jax import lax
from jax.experimental import pallas as pl
from jax.experimental.pallas import tpu as pltpu

_TM = 200
_NR = 3


def _body(x_ref, adj_ref, w1_ref, b1_ref, w2_ref, b2_ref, out_ref,
          s1_ref, s2_ref, s2b_ref, rb_ref, abuf_ref, sem_ref):
    n = x_ref.shape[0]
    nt = n // _TM
    nr = rb_ref.shape[0]
    nfetch = 2 * nt - nr
    total = 2 * nt

    s1_ref[...] = jnp.dot(
        x_ref[...].astype(jnp.bfloat16), w1_ref[...],
        preferred_element_type=jnp.float32).astype(jnp.bfloat16)

    def _copy(k, slot):
        tile = lax.rem(k, nt)
        return pltpu.make_async_copy(
            adj_ref.at[pl.ds(tile * _TM, _TM), :], abuf_ref.at[slot],
            sem_ref.at[slot])

    _copy(0, 0).start()

    def _loop(i, carry):
        consuming = i < nfetch
        slot = lax.rem(i, 2)
        nxt = lax.rem(i + 1, 2)

        @pl.when(i + 1 < nfetch)
        def _():
            _copy(i + 1, nxt).start()

        @pl.when(consuming)
        def _():
            _copy(i, slot).wait()

        @pl.when(i < nt)
        def _():
            a = abuf_ref[slot].astype(jnp.bfloat16)
            row = i * _TM
            acc = jnp.dot(a, s1_ref[...], preferred_element_type=jnp.float32)
            h = jnp.maximum(acc + b1_ref[...], 0.0).astype(jnp.bfloat16)
            s2_ref[pl.ds(row, _TM), :] = jnp.dot(
                h, w2_ref[...], preferred_element_type=jnp.float32)

            @pl.when(i >= nt - nr)
            def _():
                rb_ref[i - (nt - nr)] = a

        @pl.when(i == nt)
        def _():
            s2b_ref[...] = s2_ref[...].astype(jnp.bfloat16)

        @pl.when((i >= nt) & consuming)
        def _():
            a = abuf_ref[slot].astype(jnp.bfloat16)
            row = (i - nt) * _TM
            acc = jnp.dot(a, s2b_ref[...], preferred_element_type=jnp.float32)
            out_ref[pl.ds(row, _TM), :] = acc + b2_ref[...]

        @pl.when(~consuming)
        def _():
            j = i - nfetch
            a = rb_ref[j]
            acc = jnp.dot(a, s2b_ref[...], preferred_element_type=jnp.float32)
            out_ref[pl.ds((nt - nr + j) * _TM, _TM), :] = acc + b2_ref[...]

        return carry

    lax.fori_loop(0, total, _loop, 0)


def kernel(x, adj, W1, b1, W2, b2):
    n, nfeat = x.shape
    nhid = W1.shape[1]
    nout = W2.shape[1]
    w1b = W1.astype(jnp.bfloat16)
    w2b = W2.astype(jnp.bfloat16)
    b1r = b1.reshape(1, nhid)
    b2r = b2.reshape(1, nout)
    nt = n // _TM
    nr = min(_NR, nt - 1)

    out = pl.pallas_call(
        _body,
        in_specs=[
            pl.BlockSpec(memory_space=pltpu.VMEM),
            pl.BlockSpec(memory_space=pl.ANY),
            pl.BlockSpec(memory_space=pltpu.VMEM),
            pl.BlockSpec(memory_space=pltpu.VMEM),
            pl.BlockSpec(memory_space=pltpu.VMEM),
            pl.BlockSpec(memory_space=pltpu.VMEM),
        ],
        out_specs=pl.BlockSpec(memory_space=pltpu.VMEM),
        out_shape=jax.ShapeDtypeStruct((n, nout), jnp.float32),
        scratch_shapes=[
            pltpu.VMEM((n, nhid), jnp.bfloat16),
            pltpu.VMEM((n, nout), jnp.float32),
            pltpu.VMEM((n, nout), jnp.bfloat16),
            pltpu.VMEM((nr, _TM, n), jnp.bfloat16),
            pltpu.VMEM((2, _TM, n), jnp.float32),
            pltpu.SemaphoreType.DMA((2,)),
        ],
    )(x, adj, w1b, b1r, w2b, b2r)

    return out
